# Initial kernel scaffold; baseline (speedup 1.0000x reference)
import jax
import jax.numpy as jnp
from jax.experimental import pallas as pl


def kernel(grid, indices, coords, noise, W1, b1, W2):
    raise NotImplementedError("write your pallas kernel here")


# speedup vs baseline: 1.8228x; 1.8228x over previous
import functools

import jax
import jax.numpy as jnp
from jax import lax
from jax.experimental import pallas as pl
from jax.experimental.pallas import tpu as pltpu
from jax.experimental.pallas import tpu_sc as plsc

GRID_N = 128
NCASC = 4
SCALE_MAX = 4.0
DECAY_F = 0.95
HID = 32
G3 = GRID_N ** 3
NSAMP = 2 * (G3 // 4)

BLK = 2048

NW = 32
CHUNK_CELLS = G3 // NW
SAMP_CH = 8192
PIECE = 8192


def _dens_body(coords_ref, noise_ref, w1_ref, b1_ref, w2_ref, out_ref):
    ci = lax.broadcasted_iota(jnp.int32, (NCASC, 1, 1), 0)
    s = jnp.where(ci == 0, 0.5,
                  jnp.where(ci == 1, 1.0, jnp.where(ci == 2, 2.0, 4.0)))
    half = s / GRID_N
    x = coords_ref[...].astype(jnp.float32)
    n = noise_ref[...]
    xw = (x / (GRID_N - 1) * 2.0 - 1.0) * (s - half) + (n * 2.0 - 1.0) * half
    w1 = w1_ref[...]
    h = (xw[:, :, 0:1] * w1[0, :] + xw[:, :, 1:2] * w1[1, :]
         + xw[:, :, 2:3] * w1[2, :])
    h = jnp.maximum(h + b1_ref[...], 0.0)
    z = jnp.sum(h * w2_ref[...], axis=-1)
    out_ref[...] = jnp.maximum(z, 0.0) + jnp.log1p(jnp.exp(-jnp.abs(z)))


def _tc_density(coords, noise, W1, b1, w2v):
    return pl.pallas_call(
        _dens_body,
        grid=(NSAMP // BLK,),
        in_specs=[
            pl.BlockSpec((NCASC, BLK, 3), lambda m: (0, m, 0)),
            pl.BlockSpec((NCASC, BLK, 3), lambda m: (0, m, 0)),
            pl.BlockSpec((3, HID), lambda m: (0, 0)),
            pl.BlockSpec((HID,), lambda m: (0,)),
            pl.BlockSpec((HID,), lambda m: (0,)),
        ],
        out_specs=pl.BlockSpec((NCASC, BLK), lambda m: (0, m)),
        out_shape=jax.ShapeDtypeStruct((NCASC, NSAMP), jnp.float32),
    )(coords, noise, W1, b1, w2v)


def _sc_body(grid_hbm, idx_hbm, den_hbm, out_hbm, tmp_v, idx_v, den_v, gp_v):
    wid = lax.axis_index("s") * 2 + lax.axis_index("c")
    base = wid * CHUNK_CELLS
    for c in range(NCASC):
        def zbody(i, carry):
            tmp_v[pl.ds(i * 16, 16)] = jnp.zeros((16,), jnp.float32)
            return carry
        lax.fori_loop(0, CHUNK_CELLS // 16, zbody, 0)

        def chbody(k, carry):
            off = k * SAMP_CH
            pltpu.sync_copy(idx_hbm.at[c, pl.ds(off, SAMP_CH)], idx_v)
            pltpu.sync_copy(den_hbm.at[c, pl.ds(off, SAMP_CH)], den_v)

            def vbody(j, inner):
                iv = idx_v[pl.ds(j * 16, 16)]
                dv = den_v[pl.ds(j * 16, 16)]
                rel = iv - base
                m = (rel >= 0) & (rel < CHUNK_CELLS)
                plsc.store_scatter(tmp_v, [rel], dv, mask=m)
                return inner
            lax.fori_loop(0, SAMP_CH // 16, vbody, 0)
            return carry
        lax.fori_loop(0, NSAMP // SAMP_CH, chbody, 0)

        def mbody(p, carry):
            pltpu.sync_copy(grid_hbm.at[c, pl.ds(base + p * PIECE, PIECE)], gp_v)

            def jbody(j, inner):
                g = gp_v[pl.ds(j * 16, 16)]
                t = tmp_v[pl.ds(p * PIECE + j * 16, 16)]
                gp_v[pl.ds(j * 16, 16)] = jnp.where(
                    g < 0.0, g, jnp.maximum(g * DECAY_F, t))
                return inner
            lax.fori_loop(0, PIECE // 16, jbody, 0)
            pltpu.sync_copy(gp_v, out_hbm.at[c, pl.ds(base + p * PIECE, PIECE)])
            return carry
        lax.fori_loop(0, CHUNK_CELLS // PIECE, mbody, 0)


@functools.cache
def _sc_scatter_merge():
    return pl.kernel(
        _sc_body,
        out_type=jax.ShapeDtypeStruct((NCASC, G3), jnp.float32),
        mesh=plsc.VectorSubcoreMesh(
            core_axis_name="c", subcore_axis_name="s",
            num_cores=2, num_subcores=16),
        scratch_types=[
            pltpu.VMEM((CHUNK_CELLS,), jnp.float32),
            pltpu.VMEM((SAMP_CH,), jnp.int32),
            pltpu.VMEM((SAMP_CH,), jnp.float32),
            pltpu.VMEM((PIECE,), jnp.float32),
        ],
        compiler_params=pltpu.CompilerParams(needs_layout_passes=False),
    )


def kernel(grid, indices, coords, noise, W1, b1, W2):
    dens = _tc_density(coords, noise, W1, b1, W2.reshape(-1))
    return _sc_scatter_merge()(grid, indices, dens)


# speedup vs baseline: 2.0382x; 1.1182x over previous
import functools

import jax
import jax.numpy as jnp
from jax import lax
from jax.experimental import pallas as pl
from jax.experimental.pallas import tpu as pltpu
from jax.experimental.pallas import tpu_sc as plsc

GRID_N = 128
NCASC = 4
SCALE_MAX = 4.0
DECAY_F = 0.95
HID = 32
G3 = GRID_N ** 3
NSAMP = 2 * (G3 // 4)

BLK = 2048

NW = 32
CHUNK_CELLS = G3 // NW
SAMP_CH = 8192
PIECE = 8192


def _dens_body(coords_ref, noise_ref, w1_ref, b1_ref, w2_ref, out_ref):
    ci = lax.broadcasted_iota(jnp.int32, (NCASC, 1, 1), 0)
    s = jnp.where(ci == 0, 0.5,
                  jnp.where(ci == 1, 1.0, jnp.where(ci == 2, 2.0, 4.0)))
    half = s / GRID_N
    x = coords_ref[...].astype(jnp.float32)
    n = noise_ref[...]
    xw = (x / (GRID_N - 1) * 2.0 - 1.0) * (s - half) + (n * 2.0 - 1.0) * half
    w1 = w1_ref[...]
    h = (xw[:, :, 0:1] * w1[0, :] + xw[:, :, 1:2] * w1[1, :]
         + xw[:, :, 2:3] * w1[2, :])
    h = jnp.maximum(h + b1_ref[...], 0.0)
    z = jnp.sum(h * w2_ref[...], axis=-1)
    out_ref[...] = jnp.maximum(z, 0.0) + jnp.log1p(jnp.exp(-jnp.abs(z)))


def _tc_density(coords, noise, W1, b1, w2v):
    return pl.pallas_call(
        _dens_body,
        grid=(NSAMP // BLK,),
        in_specs=[
            pl.BlockSpec((NCASC, BLK, 3), lambda m: (0, m, 0)),
            pl.BlockSpec((NCASC, BLK, 3), lambda m: (0, m, 0)),
            pl.BlockSpec((3, HID), lambda m: (0, 0)),
            pl.BlockSpec((HID,), lambda m: (0,)),
            pl.BlockSpec((HID,), lambda m: (0,)),
        ],
        out_specs=pl.BlockSpec((NCASC, BLK), lambda m: (0, m)),
        out_shape=jax.ShapeDtypeStruct((NCASC, NSAMP), jnp.float32),
    )(coords, noise, W1, b1, w2v)


def _sc_body(grid_hbm, idx_hbm, den_hbm, out_hbm, tmp_v, idx_v, den_v, gp_v,
             sem_a, sem_b):
    wid = lax.axis_index("s") * 2 + lax.axis_index("c")
    base = wid * CHUNK_CELLS
    sems = (sem_a, sem_b)
    nch = NSAMP // SAMP_CH

    def start_pair(c, b, off):
        pltpu.make_async_copy(
            idx_hbm.at[c, pl.ds(off, SAMP_CH)], idx_v.at[b], sems[b]).start()
        pltpu.make_async_copy(
            den_hbm.at[c, pl.ds(off, SAMP_CH)], den_v.at[b], sems[b]).start()

    def wait_pair(c, b):
        pltpu.make_async_copy(
            idx_hbm.at[c, pl.ds(0, SAMP_CH)], idx_v.at[b], sems[b]).wait()
        pltpu.make_async_copy(
            den_hbm.at[c, pl.ds(0, SAMP_CH)], den_v.at[b], sems[b]).wait()

    for c in range(NCASC):
        def zbody(i, carry):
            tmp_v[pl.ds(i * 16, 16)] = jnp.zeros((16,), jnp.float32)
            return carry
        lax.fori_loop(0, CHUNK_CELLS // 16, zbody, 0, unroll=8)

        for b in range(2):
            start_pair(c, b, b * SAMP_CH)

        def pair_body(k2, carry):
            for b in range(2):
                k = k2 * 2 + b
                wait_pair(c, b)

                def vbody(j, inner):
                    iv = idx_v[b, pl.ds(j * 16, 16)]
                    dv = den_v[b, pl.ds(j * 16, 16)]
                    rel = iv - base
                    m = rel.astype(jnp.uint32) < jnp.uint32(CHUNK_CELLS)
                    plsc.store_scatter(tmp_v, [rel], dv, mask=m)
                    return inner
                lax.fori_loop(0, SAMP_CH // 16, vbody, 0, unroll=8)
                off_n = jnp.minimum((k + 2) * SAMP_CH, NSAMP - SAMP_CH)
                start_pair(c, b, off_n)
            return carry
        lax.fori_loop(0, nch // 2, pair_body, 0)
        for b in range(2):
            wait_pair(c, b)

        def mbody(p, carry):
            pltpu.sync_copy(grid_hbm.at[c, pl.ds(base + p * PIECE, PIECE)], gp_v)

            def jbody(j, inner):
                g = gp_v[pl.ds(j * 16, 16)]
                t = tmp_v[pl.ds(p * PIECE + j * 16, 16)]
                gp_v[pl.ds(j * 16, 16)] = jnp.where(
                    g < 0.0, g, jnp.maximum(g * DECAY_F, t))
                return inner
            lax.fori_loop(0, PIECE // 16, jbody, 0, unroll=8)
            pltpu.sync_copy(gp_v, out_hbm.at[c, pl.ds(base + p * PIECE, PIECE)])
            return carry
        lax.fori_loop(0, CHUNK_CELLS // PIECE, mbody, 0)


@functools.cache
def _sc_scatter_merge():
    return pl.kernel(
        _sc_body,
        out_type=jax.ShapeDtypeStruct((NCASC, G3), jnp.float32),
        mesh=plsc.VectorSubcoreMesh(
            core_axis_name="c", subcore_axis_name="s",
            num_cores=2, num_subcores=16),
        scratch_types=[
            pltpu.VMEM((CHUNK_CELLS,), jnp.float32),
            pltpu.VMEM((2, SAMP_CH), jnp.int32),
            pltpu.VMEM((2, SAMP_CH), jnp.float32),
            pltpu.VMEM((PIECE,), jnp.float32),
            pltpu.SemaphoreType.DMA,
            pltpu.SemaphoreType.DMA,
        ],
        compiler_params=pltpu.CompilerParams(needs_layout_passes=False),
    )


def kernel(grid, indices, coords, noise, W1, b1, W2):
    dens = _tc_density(coords, noise, W1, b1, W2.reshape(-1))
    return _sc_scatter_merge()(grid, indices, dens)


# speedup vs baseline: 6.2434x; 3.0632x over previous
import functools

import jax
import jax.numpy as jnp
from jax import lax
from jax.experimental import pallas as pl
from jax.experimental.pallas import tpu as pltpu
from jax.experimental.pallas import tpu_sc as plsc

GRID_N = 128
NCASC = 4
DECAY_F = 0.95
HID = 32
G3 = GRID_N ** 3
NSAMP = 2 * (G3 // 4)

BLKL = 8192

NW = 32
CHUNK_CELLS = G3 // NW
SAMP_CH = 4096
NBUF = 4
PIECE = 8192


def _dens_body(ct_x, ct_y, ct_z, nt_x, nt_y, nt_z, w1_ref, b1_ref, w2_ref,
               out_ref):
    ci = lax.broadcasted_iota(jnp.int32, (NCASC, 1), 0)
    s = jnp.where(ci == 0, 0.5,
                  jnp.where(ci == 1, 1.0, jnp.where(ci == 2, 2.0, 4.0)))
    half = s / GRID_N
    amp = s - half

    def jitter(cr, nr):
        x = cr[0].astype(jnp.float32)
        return (x / (GRID_N - 1) * 2.0 - 1.0) * amp + (nr[0] * 2.0 - 1.0) * half

    X = jitter(ct_x, nt_x)
    Y = jitter(ct_y, nt_y)
    Z = jitter(ct_z, nt_z)
    acc = None
    for j in range(HID):
        h = X * w1_ref[0, j] + Y * w1_ref[1, j] + Z * w1_ref[2, j] + b1_ref[j]
        h = jnp.maximum(h, 0.0) * w2_ref[j]
        acc = h if acc is None else acc + h
    out_ref[...] = jnp.maximum(acc, 0.0) + jnp.log1p(jnp.exp(-jnp.abs(acc)))


def _tc_density(ct, nt, W1, b1, w2v):
    comp_spec = lambda k: pl.BlockSpec((1, NCASC, BLKL), lambda m, _k=k: (_k, 0, m))
    smem = pl.BlockSpec(memory_space=pltpu.SMEM)
    return pl.pallas_call(
        _dens_body,
        grid=(NSAMP // BLKL,),
        in_specs=[comp_spec(0), comp_spec(1), comp_spec(2),
                  comp_spec(0), comp_spec(1), comp_spec(2),
                  smem, smem, smem],
        out_specs=pl.BlockSpec((NCASC, BLKL), lambda m: (0, m)),
        out_shape=jax.ShapeDtypeStruct((NCASC, NSAMP), jnp.float32),
    )(ct, ct, ct, nt, nt, nt, W1, b1, w2v)


def _sc_body(grid_hbm, idx_hbm, den_hbm, out_hbm, tmp_v, idx_v, den_v, gp_v,
             sems, gsem, osem):
    wid = lax.axis_index("s") * 2 + lax.axis_index("c")
    base = wid * CHUNK_CELLS
    nch = NSAMP // SAMP_CH

    def start_pair(c, b, off):
        pltpu.make_async_copy(
            idx_hbm.at[c, pl.ds(off, SAMP_CH)], idx_v.at[b], sems.at[b]).start()
        pltpu.make_async_copy(
            den_hbm.at[c, pl.ds(off, SAMP_CH)], den_v.at[b], sems.at[b]).start()

    def wait_pair(c, b):
        pltpu.make_async_copy(
            idx_hbm.at[c, pl.ds(0, SAMP_CH)], idx_v.at[b], sems.at[b]).wait()
        pltpu.make_async_copy(
            den_hbm.at[c, pl.ds(0, SAMP_CH)], den_v.at[b], sems.at[b]).wait()

    def prime(c):
        for b in range(NBUF):
            start_pair(c, b, b * SAMP_CH)

    npieces = CHUNK_CELLS // PIECE
    prime(0)

    def casc_body(c, carry0):
        def zbody(i, carry):
            tmp_v[pl.ds(i * 16, 16)] = jnp.zeros((16,), jnp.float32)
            return carry
        lax.fori_loop(0, CHUNK_CELLS // 16, zbody, 0, unroll=8)

        def ring_body(kb, carry):
            for b in range(NBUF):
                k = kb * NBUF + b
                wait_pair(c, b)

                def vbody(j, inner):
                    iv = idx_v[b, pl.ds(j * 16, 16)]
                    dv = den_v[b, pl.ds(j * 16, 16)]
                    rel = iv - base
                    m = rel.astype(jnp.uint32) < jnp.uint32(CHUNK_CELLS)
                    plsc.store_scatter(tmp_v, [rel], dv, mask=m)
                    return inner
                lax.fori_loop(0, SAMP_CH // 16, vbody, 0, unroll=8)
                off_n = jnp.minimum((k + NBUF) * SAMP_CH, NSAMP - SAMP_CH)
                start_pair(c, b, off_n)
            return carry
        lax.fori_loop(0, nch // NBUF, ring_body, 0)
        for b in range(NBUF):
            wait_pair(c, b)

        @pl.when(c < NCASC - 1)
        def _():
            prime(c + 1)

        pltpu.make_async_copy(
            grid_hbm.at[c, pl.ds(base, PIECE)], gp_v.at[0], gsem).start()

        def merge_body(p2, carry):
            for gb in range(2):
                p = p2 * 2 + gb
                pltpu.make_async_copy(
                    grid_hbm.at[c, pl.ds(0, PIECE)], gp_v.at[gb], gsem).wait()

                @pl.when(p < npieces - 1)
                def _():
                    pltpu.make_async_copy(
                        grid_hbm.at[c, pl.ds(base + (p + 1) * PIECE, PIECE)],
                        gp_v.at[1 - gb], gsem).start()

                def jbody(j, inner):
                    g = gp_v[gb, pl.ds(j * 16, 16)]
                    t = tmp_v[pl.ds(p * PIECE + j * 16, 16)]
                    tmp_v[pl.ds(p * PIECE + j * 16, 16)] = jnp.where(
                        g < 0.0, g, jnp.maximum(g * DECAY_F, t))
                    return inner
                lax.fori_loop(0, PIECE // 16, jbody, 0, unroll=8)
                pltpu.make_async_copy(
                    tmp_v.at[pl.ds(p * PIECE, PIECE)],
                    out_hbm.at[c, pl.ds(base + p * PIECE, PIECE)], osem).start()
            return carry
        lax.fori_loop(0, npieces // 2, merge_body, 0)

        def drain_body(p, carry):
            pltpu.make_async_copy(
                tmp_v.at[pl.ds(0, PIECE)],
                out_hbm.at[c, pl.ds(base, PIECE)], osem).wait()
            return carry
        lax.fori_loop(0, npieces, drain_body, 0)
        return carry0

    lax.fori_loop(0, NCASC, casc_body, 0)


@functools.cache
def _sc_scatter_merge():
    return pl.kernel(
        _sc_body,
        out_type=jax.ShapeDtypeStruct((NCASC, G3), jnp.float32),
        mesh=plsc.VectorSubcoreMesh(
            core_axis_name="c", subcore_axis_name="s",
            num_cores=2, num_subcores=16),
        scratch_types=[
            pltpu.VMEM((CHUNK_CELLS,), jnp.float32),
            pltpu.VMEM((NBUF, SAMP_CH), jnp.int32),
            pltpu.VMEM((NBUF, SAMP_CH), jnp.float32),
            pltpu.VMEM((2, PIECE), jnp.float32),
            pltpu.SemaphoreType.DMA((NBUF,)),
            pltpu.SemaphoreType.DMA,
            pltpu.SemaphoreType.DMA,
        ],
        compiler_params=pltpu.CompilerParams(needs_layout_passes=False),
    )


def kernel(grid, indices, coords, noise, W1, b1, W2):
    ct = coords.transpose(2, 0, 1)
    nt = noise.transpose(2, 0, 1)
    dens = _tc_density(ct, nt, W1, b1, W2.reshape(-1))
    return _sc_scatter_merge()(grid, indices, dens)


# speedup vs baseline: 10.9703x; 1.7571x over previous
import functools

import jax
import jax.numpy as jnp
from jax import lax
from jax.experimental import pallas as pl
from jax.experimental.pallas import tpu as pltpu
from jax.experimental.pallas import tpu_sc as plsc

GRID_N = 128
NCASC = 4
DECAY_F = 0.95
HID = 32
G3 = GRID_N ** 3
NSAMP = 2 * (G3 // 4)

BLKL = 8192

NW = 32
CHUNK_CELLS = G3 // NW
SAMP_CH = 8192
NBUF = 2
PIECE = 8192


def _dens_body(s, half, ct_ref, nt_ref, w1_ref, b1_ref, w2_ref, out_ref):
    amp = s - half
    x = ct_ref[...].astype(jnp.float32)
    xw = (x / (GRID_N - 1) * 2.0 - 1.0) * amp + (nt_ref[...] * 2.0 - 1.0) * half
    X = xw[0:1, :]
    Y = xw[1:2, :]
    Z = xw[2:3, :]
    acc = None
    for j in range(HID):
        h = X * w1_ref[0, j] + Y * w1_ref[1, j] + Z * w1_ref[2, j] + b1_ref[j]
        h = jnp.maximum(h, 0.0) * w2_ref[j]
        acc = h if acc is None else acc + h
    out_ref[...] = jnp.maximum(acc, 0.0) + jnp.log1p(jnp.exp(-jnp.abs(acc)))


def _tc_density(c, ct, nt, W1, b1, w2v):
    s = min(2.0 ** (c - 1), 4.0)
    half = s / GRID_N
    smem = pl.BlockSpec(memory_space=pltpu.SMEM)
    return pl.pallas_call(
        functools.partial(_dens_body, s, half),
        grid=(NSAMP // BLKL,),
        in_specs=[
            pl.BlockSpec((3, BLKL), lambda m: (0, m)),
            pl.BlockSpec((3, BLKL), lambda m: (0, m)),
            smem, smem, smem,
        ],
        out_specs=pl.BlockSpec((1, BLKL), lambda m: (0, m)),
        out_shape=jax.ShapeDtypeStruct((1, NSAMP), jnp.float32),
    )(ct, nt, W1, b1, w2v)


def _sc_body(c, grid_hbm, idx_hbm, den_hbm, out_hbm, tmp_v, idx_v, den_v,
             gp_v, sems, gsem, osem):
    wid = lax.axis_index("s") * 2 + lax.axis_index("c")
    base = wid * CHUNK_CELLS
    nch = NSAMP // SAMP_CH
    npieces = CHUNK_CELLS // PIECE

    def start_pair(b, off):
        pltpu.make_async_copy(
            idx_hbm.at[c, pl.ds(off, SAMP_CH)], idx_v.at[b], sems.at[b]).start()
        pltpu.make_async_copy(
            den_hbm.at[0, pl.ds(off, SAMP_CH)], den_v.at[b], sems.at[b]).start()

    def wait_pair(b):
        pltpu.make_async_copy(
            idx_hbm.at[c, pl.ds(0, SAMP_CH)], idx_v.at[b], sems.at[b]).wait()
        pltpu.make_async_copy(
            den_hbm.at[0, pl.ds(0, SAMP_CH)], den_v.at[b], sems.at[b]).wait()

    for b in range(NBUF):
        start_pair(b, b * SAMP_CH)

    def zbody(i, carry):
        tmp_v[pl.ds(i * 16, 16)] = jnp.zeros((16,), jnp.float32)
        return carry
    lax.fori_loop(0, CHUNK_CELLS // 16, zbody, 0, unroll=8)

    def ring_body(kb, carry):
        for b in range(NBUF):
            k = kb * NBUF + b
            wait_pair(b)

            UNR = 16

            def vbody(j, inner):
                j0 = j * UNR
                ivs = [idx_v[b, pl.ds((j0 + u) * 16, 16)] for u in range(UNR)]
                dvs = [den_v[b, pl.ds((j0 + u) * 16, 16)] for u in range(UNR)]
                rels = [iv - base for iv in ivs]
                ms = [r.astype(jnp.uint32) < jnp.uint32(CHUNK_CELLS)
                      for r in rels]
                for u in range(UNR):
                    plsc.store_scatter(tmp_v, [rels[u]], dvs[u], mask=ms[u])
                return inner
            lax.fori_loop(0, SAMP_CH // 16 // UNR, vbody, 0)
            off_n = jnp.minimum((k + NBUF) * SAMP_CH, NSAMP - SAMP_CH)
            start_pair(b, off_n)
        return carry
    lax.fori_loop(0, nch // NBUF, ring_body, 0)
    for b in range(NBUF):
        wait_pair(b)

    pltpu.make_async_copy(
        grid_hbm.at[c, pl.ds(base, PIECE)], gp_v.at[0], gsem).start()

    def merge_body(p2, carry):
        for gb in range(2):
            p = p2 * 2 + gb
            pltpu.make_async_copy(
                grid_hbm.at[c, pl.ds(0, PIECE)], gp_v.at[gb], gsem).wait()

            @pl.when(p < npieces - 1)
            def _():
                pltpu.make_async_copy(
                    grid_hbm.at[c, pl.ds(base + (p + 1) * PIECE, PIECE)],
                    gp_v.at[1 - gb], gsem).start()

            def jbody(j, inner):
                g = gp_v[gb, pl.ds(j * 16, 16)]
                t = tmp_v[pl.ds(p * PIECE + j * 16, 16)]
                tmp_v[pl.ds(p * PIECE + j * 16, 16)] = jnp.where(
                    g < 0.0, g, jnp.maximum(g * DECAY_F, t))
                return inner
            lax.fori_loop(0, PIECE // 16, jbody, 0, unroll=8)
            pltpu.make_async_copy(
                tmp_v.at[pl.ds(p * PIECE, PIECE)],
                out_hbm.at[pl.ds(base + p * PIECE, PIECE)], osem).start()
        return carry
    lax.fori_loop(0, npieces // 2, merge_body, 0)

    def drain_body(p, carry):
        pltpu.make_async_copy(
            tmp_v.at[pl.ds(0, PIECE)],
            out_hbm.at[pl.ds(base, PIECE)], osem).wait()
        return carry
    lax.fori_loop(0, npieces, drain_body, 0)


@functools.cache
def _sc_scatter_merge(c):
    return pl.kernel(
        functools.partial(_sc_body, c),
        out_type=jax.ShapeDtypeStruct((G3,), jnp.float32),
        mesh=plsc.VectorSubcoreMesh(
            core_axis_name="c", subcore_axis_name="s",
            num_cores=2, num_subcores=16),
        scratch_types=[
            pltpu.VMEM((CHUNK_CELLS,), jnp.float32),
            pltpu.VMEM((NBUF, SAMP_CH), jnp.int32),
            pltpu.VMEM((NBUF, SAMP_CH), jnp.float32),
            pltpu.VMEM((2, PIECE), jnp.float32),
            pltpu.SemaphoreType.DMA((NBUF,)),
            pltpu.SemaphoreType.DMA,
            pltpu.SemaphoreType.DMA,
        ],
        compiler_params=pltpu.CompilerParams(needs_layout_passes=False),
    )


def kernel(grid, indices, coords, noise, W1, b1, W2):
    w2v = W2.reshape(-1)
    outs = []
    for c in range(NCASC):
        ct = coords[c].transpose(1, 0)
        nt = noise[c].transpose(1, 0)
        dens = _tc_density(c, ct, nt, W1, b1, w2v)
        outs.append(_sc_scatter_merge(c)(grid, indices, dens))
    return jnp.stack(outs)
